# baseline (speedup 1.0000x reference)
import functools

import jax
import jax.numpy as jnp
from jax.experimental import pallas as pl


def _loss_block(ps_ref, pe_ref, pc_ref, gt_ref, out_ref):
    b = pl.program_id(0)
    BB, Pp, _, T, F = ps_ref.shape
    N = gt_ref.shape[1]

    main_b = jnp.float32(0.0)
    npos_b = jnp.float32(0.0)
    for bb in range(BB):
        g = gt_ref[bb]
        s = jnp.clip(g[:, 0:1], 0.0, 1.0)
        e = jnp.clip(g[:, 1:2], 0.0, 1.0)

        lane = jax.lax.broadcasted_iota(
            jnp.int32, (1, F), 1).astype(jnp.float32)
        left = lane * (1.0 / F)
        right = left + (1.0 / F)
        ov = jnp.clip(jnp.minimum(e, right) - jnp.maximum(s, left), 0.0, None)
        not_skip = jnp.sum(ov) > 0.0
        n_col = jax.lax.broadcasted_iota(jnp.int32, (N, F), 0)
        s_b = jnp.broadcast_to(s, (N, F))
        e_b = jnp.broadcast_to(e, (N, F))

        for p in range(Pp):
            m = jnp.max(ov, axis=0, keepdims=True)
            idx = jnp.min(jnp.where(ov == m, n_col, N), axis=0, keepdims=True)
            oh = n_col == idx
            ts = jnp.sum(jnp.where(oh, s_b, 0.0), axis=0, keepdims=True)
            te = jnp.sum(jnp.where(oh, e_b, 0.0), axis=0, keepdims=True)
            pos = (m > 0.0) & not_skip
            ov = jnp.where(oh, -1.0, ov)

            z = pos.astype(jnp.float32)
            rw = 5.0 * z
            aw = 0.5 + 0.5 * z

            ps = ps_ref[bb, p, 0]
            pe = pe_ref[bb, p, 0]
            pc = pc_ref[bb, p, 0]
            d1 = jnp.abs(ps - ts)
            m1 = jnp.minimum(d1, 1.0)
            d2 = jnp.abs(pe - te)
            m2 = jnp.minimum(d2, 1.0)
            sl = m1 * (d1 - 0.5 * m1) + m2 * (d2 - 0.5 * m2)
            sp = jnp.maximum(pc, 0.0) + jnp.log1p(jnp.exp(-jnp.abs(pc)))
            contrib = rw * sl + aw * sp - z * pc
            main_b += jnp.sum(contrib)
            npos_b += jnp.float32(T) * jnp.sum(z)

    blk = jnp.concatenate(
        [jnp.full((1, 128), main_b, jnp.float32),
         jnp.full((1, 128), npos_b, jnp.float32),
         jnp.zeros((6, 128), jnp.float32)], axis=0)

    @pl.when(b == 0)
    def _():
        out_ref[...] = blk

    @pl.when(b != 0)
    def _():
        out_ref[...] = out_ref[...] + blk


@functools.partial(jax.jit, static_argnames=())
def kernel(raw_preds, gt_boxes):
    B, Pp, C, T, F = raw_preds.shape
    N = gt_boxes.shape[1]
    BB = 8
    out = pl.pallas_call(
        _loss_block,
        grid=(B // BB,),
        in_specs=[
            pl.BlockSpec((BB, Pp, 1, T, F), lambda b: (b, 0, 0, 0, 0)),
            pl.BlockSpec((BB, Pp, 1, T, F), lambda b: (b, 0, 1, 0, 0)),
            pl.BlockSpec((BB, Pp, 1, T, F), lambda b: (b, 0, 2, 0, 0)),
            pl.BlockSpec((BB, N, 2), lambda b: (b, 0, 0)),
        ],
        out_specs=pl.BlockSpec((8, 128), lambda b: (0, 0)),
        out_shape=jax.ShapeDtypeStruct((8, 128), jnp.float32),
    )(raw_preds, raw_preds, raw_preds, gt_boxes)
    main = out[0, 0]
    n_pos = out[1, 0]
    return main / jnp.maximum(n_pos, 1.0)


# speedup vs baseline: 1.0455x; 1.0455x over previous
import functools

import jax
import jax.numpy as jnp
from jax.experimental import pallas as pl


def _loss_block(raw_ref, gt_ref, out_ref):
    b = pl.program_id(0)
    BB, Pp, _, T, F = raw_ref.shape
    N = gt_ref.shape[1]

    main_b = jnp.float32(0.0)
    npos_b = jnp.float32(0.0)
    for bb in range(BB):
        g = gt_ref[bb]
        s = jnp.clip(g[:, 0:1], 0.0, 1.0)
        e = jnp.clip(g[:, 1:2], 0.0, 1.0)

        lane = jax.lax.broadcasted_iota(
            jnp.int32, (1, F), 1).astype(jnp.float32)
        left = lane * (1.0 / F)
        right = left + (1.0 / F)
        ov = jnp.clip(jnp.minimum(e, right) - jnp.maximum(s, left), 0.0, None)
        not_skip = jnp.sum(ov) > 0.0
        n_col = jax.lax.broadcasted_iota(jnp.int32, (N, F), 0)
        s_b = jnp.broadcast_to(s, (N, F))
        e_b = jnp.broadcast_to(e, (N, F))

        for p in range(Pp):
            m = jnp.max(ov, axis=0, keepdims=True)
            idx = jnp.min(jnp.where(ov == m, n_col, N), axis=0, keepdims=True)
            oh = n_col == idx
            ts = jnp.sum(jnp.where(oh, s_b, 0.0), axis=0, keepdims=True)
            te = jnp.sum(jnp.where(oh, e_b, 0.0), axis=0, keepdims=True)
            pos = (m > 0.0) & not_skip
            ov = jnp.where(oh, -1.0, ov)

            z = pos.astype(jnp.float32)
            rw = 5.0 * z
            aw = 0.5 + 0.5 * z

            ps = raw_ref[bb, p, 0]
            pe = raw_ref[bb, p, 1]
            pc = raw_ref[bb, p, 2]
            d1 = jnp.abs(ps - ts)
            m1 = jnp.minimum(d1, 1.0)
            d2 = jnp.abs(pe - te)
            m2 = jnp.minimum(d2, 1.0)
            sl = m1 * (d1 - 0.5 * m1) + m2 * (d2 - 0.5 * m2)
            sp = jnp.maximum(pc, 0.0) + jnp.log1p(jnp.exp(-jnp.abs(pc)))
            contrib = rw * sl + aw * sp - z * pc
            main_b += jnp.sum(contrib)
            npos_b += jnp.float32(T) * jnp.sum(z)

    blk = jnp.concatenate(
        [jnp.full((1, 128), main_b, jnp.float32),
         jnp.full((1, 128), npos_b, jnp.float32),
         jnp.zeros((6, 128), jnp.float32)], axis=0)

    @pl.when(b == 0)
    def _():
        out_ref[...] = blk

    @pl.when(b != 0)
    def _():
        out_ref[...] = out_ref[...] + blk


@functools.partial(jax.jit, static_argnames=())
def kernel(raw_preds, gt_boxes):
    B, Pp, C, T, F = raw_preds.shape
    N = gt_boxes.shape[1]
    BB = 4
    out = pl.pallas_call(
        _loss_block,
        grid=(B // BB,),
        in_specs=[
            pl.BlockSpec((BB, Pp, C, T, F), lambda b: (b, 0, 0, 0, 0)),
            pl.BlockSpec((BB, N, 2), lambda b: (b, 0, 0)),
        ],
        out_specs=pl.BlockSpec((8, 128), lambda b: (0, 0)),
        out_shape=jax.ShapeDtypeStruct((8, 128), jnp.float32),
    )(raw_preds, gt_boxes)
    main = out[0, 0]
    n_pos = out[1, 0]
    return main / jnp.maximum(n_pos, 1.0)
